# baseline (speedup 1.0000x reference)
import jax
import jax.numpy as jnp
from jax import lax
from jax.experimental import pallas as pl
from jax.experimental.pallas import tpu as pltpu
from jax.experimental.pallas import tpu_sc as plsc

NUM_COLS = 2048
NUM_ROWS = 16384
NC = 2
NS = 16
L = 16
NW = NC * NS
ROWS_PER_W = NUM_ROWS // NW
R = 2
BLK = R * NUM_COLS
NBLK = ROWS_PER_W // R
CHUNKS = NUM_COLS // L
JU = 8


NDB = 8


def _body(in_hbm, perm_hbm, out_hbm, perm_v, *rest):
    in_bufs = rest[0:NDB]
    out_bufs = rest[NDB:2 * NDB]
    sem_in = rest[2 * NDB:3 * NDB]
    sem_out = rest[3 * NDB:4 * NDB]

    wid = lax.axis_index("c") * NS + lax.axis_index("s")
    pltpu.sync_copy(perm_hbm, perm_v)

    def in_desc(b, p):
        return pltpu.make_async_copy(
            in_hbm.at[pl.ds((wid + NW * b) * R, R), :], in_bufs[p], sem_in[p])

    def out_desc(b, p):
        return pltpu.make_async_copy(
            out_bufs[p], out_hbm.at[pl.ds((wid + NW * b) * R, R), :], sem_out[p])

    for p in range(NDB):
        in_desc(p, p).start()

    def pair(i, carry):
        for p in range(NDB):
            b = NDB * i + p
            in_desc(b, p).wait()

            @pl.when(i >= 1)
            def _():
                out_desc(b - NDB, p).wait()

            in_v = in_bufs[p]
            out_v = out_bufs[p]

            @plsc.parallel_loop(0, CHUNKS, unroll=JU)
            def _(j):
                j0 = j * L
                idx = perm_v[pl.ds(j0, L)]
                zeros = idx - idx
                for r in range(R):
                    vals = plsc.load_gather(in_v, [zeros + r, idx])
                    out_v[r, pl.ds(j0, L)] = vals
            out_desc(b, p).start()

            @pl.when(i < NBLK // NDB - 1)
            def _():
                in_desc(b + NDB, p).start()
        return carry

    lax.fori_loop(0, NBLK // NDB, pair, 0)
    for p in range(NDB):
        out_desc(NBLK - NDB + p, p).wait()


@jax.jit
def _shuffle(inputs, perm_i32):
    mesh = plsc.VectorSubcoreMesh(core_axis_name="c", subcore_axis_name="s")
    return pl.kernel(
        _body,
        out_type=jax.ShapeDtypeStruct((NUM_ROWS, NUM_COLS), jnp.float32),
        mesh=mesh,
        compiler_params=pltpu.CompilerParams(needs_layout_passes=False),
        scratch_types=[
            pltpu.VMEM((NUM_COLS,), jnp.int32),
            *[pltpu.VMEM((R, NUM_COLS), jnp.float32) for _ in range(16)],
            *[pltpu.SemaphoreType.DMA for _ in range(16)],
        ],
    )(inputs, perm_i32)


def kernel(inputs, perm):
    out = _shuffle(inputs, perm.astype(jnp.int32))
    logdet = jnp.zeros((inputs.shape[0], 1), dtype=inputs.dtype)
    return (out, logdet)


# speedup vs baseline: 1.0182x; 1.0182x over previous
import jax
import jax.numpy as jnp
from jax import lax
from jax.experimental import pallas as pl
from jax.experimental.pallas import tpu as pltpu
from jax.experimental.pallas import tpu_sc as plsc

NUM_COLS = 2048
NUM_ROWS = 16384
NC = 2
NS = 16
L = 16
NW = NC * NS
ROWS_PER_W = NUM_ROWS // NW
R = 4
BLK = R * NUM_COLS
NBLK = ROWS_PER_W // R
CHUNKS = NUM_COLS // L
JU = 8


NDB = 4


def _body(in_hbm, perm_hbm, out_hbm, perm_v, *rest):
    in_bufs = rest[0:NDB]
    out_bufs = rest[NDB:2 * NDB]
    sem_in = rest[2 * NDB:3 * NDB]
    sem_out = rest[3 * NDB:4 * NDB]

    wid = lax.axis_index("c") * NS + lax.axis_index("s")
    pltpu.sync_copy(perm_hbm, perm_v)

    def in_desc(b, p):
        return pltpu.make_async_copy(
            in_hbm.at[pl.ds((wid + NW * b) * R, R), :], in_bufs[p], sem_in[p])

    def out_desc(b, p):
        return pltpu.make_async_copy(
            out_bufs[p], out_hbm.at[pl.ds((wid + NW * b) * R, R), :], sem_out[p])

    for p in range(NDB):
        in_desc(p, p).start()

    def pair(i, carry):
        for p in range(NDB):
            b = NDB * i + p
            in_desc(b, p).wait()

            @pl.when(i >= 1)
            def _():
                out_desc(b - NDB, p).wait()

            in_v = in_bufs[p]
            out_v = out_bufs[p]

            @plsc.parallel_loop(0, CHUNKS, unroll=JU)
            def _(j):
                j0 = j * L
                idx = perm_v[pl.ds(j0, L)]
                zeros = idx - idx
                for r in range(R):
                    vals = plsc.load_gather(in_v, [zeros + r, idx])
                    out_v[r, pl.ds(j0, L)] = vals
            out_desc(b, p).start()

            @pl.when(i < NBLK // NDB - 1)
            def _():
                in_desc(b + NDB, p).start()
        return carry

    lax.fori_loop(0, NBLK // NDB, pair, 0)
    for p in range(NDB):
        out_desc(NBLK - NDB + p, p).wait()


@jax.jit
def _shuffle(inputs, perm_i32):
    mesh = plsc.VectorSubcoreMesh(core_axis_name="c", subcore_axis_name="s")
    return pl.kernel(
        _body,
        out_type=jax.ShapeDtypeStruct((NUM_ROWS, NUM_COLS), jnp.float32),
        mesh=mesh,
        compiler_params=pltpu.CompilerParams(needs_layout_passes=False),
        scratch_types=[
            pltpu.VMEM((NUM_COLS,), jnp.int32),
            *[pltpu.VMEM((R, NUM_COLS), jnp.float32) for _ in range(8)],
            *[pltpu.SemaphoreType.DMA for _ in range(8)],
        ],
    )(inputs, perm_i32)


def kernel(inputs, perm):
    out = _shuffle(inputs, perm.astype(jnp.int32))
    logdet = jnp.zeros((inputs.shape[0], 1), dtype=inputs.dtype)
    return (out, logdet)
